# baseline (speedup 1.0000x reference)
import functools

import jax
import jax.numpy as jnp
from jax import lax
from jax.experimental import pallas as pl
from jax.experimental.pallas import tpu as pltpu
from jax.experimental.pallas import tpu_sc as plsc

_K = 20
_NEG = -3.0e38


def _two_sum(s, y):
    t = s + y
    bp = t - s
    e = (s - (t - bp)) + (y - bp)
    return t, e


def _comp_reduce(x, c):
    m = x.shape[0]
    while m > 1:
        h = m // 2
        a, b = x[:h], x[h:2 * h]
        t, e = _two_sum(a, b)
        c = c[:h] + c[h:2 * h] + e
        if m % 2 == 1:
            t, e2 = _two_sum(t, jnp.concatenate(
                [x[2 * h:], jnp.zeros((h - 1,) + x.shape[1:], x.dtype)],
                axis=0))
            c = c + e2
        x = t
        m = h
    return x, c


def _pairwise_dist(xt, n):
    f32 = jnp.float32
    hi = jax.lax.Precision.HIGHEST
    x2 = xt * xt
    xx_col = jnp.sum(x2, axis=1, keepdims=True)
    r_i = jax.lax.broadcasted_iota(jnp.int32, (n, n), 0)
    c_i = jax.lax.broadcasted_iota(jnp.int32, (n, n), 1)
    eye = jnp.where(r_i == c_i, 1.0, 0.0).astype(f32)
    xx_row = jax.lax.dot_general(xx_col, eye, (((0,), (0,)), ((), ())),
                                 precision=hi, preferred_element_type=f32)
    inner = jax.lax.dot_general(xt, xt, (((1,), (1,)), ((), ())),
                                preferred_element_type=f32)
    return 2.0 * inner - xx_col - xx_row, c_i


def _topk_body(xt_ref, idx_ref, d_ref):
    xt = xt_ref[0]
    n = xt.shape[0]
    b = pl.program_id(0)
    d, c_i = _pairwise_dist(xt, n)
    d_ref[...] = d
    for k in range(_K):
        d = d_ref[...]
        m = jnp.max(d, axis=1, keepdims=True)
        masked = jnp.where(d == m, c_i, n)
        amin = jnp.min(masked, axis=1, keepdims=True)
        sel = masked == amin
        d_ref[...] = jnp.where(sel, _NEG, d)
        idx_ref[0, :, k:k + 1] = amin + b * n


def _topk_indices(xt):
    B, N, C = xt.shape
    return pl.pallas_call(
        _topk_body,
        grid=(B,),
        in_specs=[pl.BlockSpec((1, N, C), lambda i: (i, 0, 0))],
        out_specs=pl.BlockSpec((1, N, _K), lambda i: (i, 0, 0)),
        out_shape=jax.ShapeDtypeStruct((B, N, _K), jnp.int32),
        scratch_shapes=[pltpu.VMEM((N, N), jnp.float32)],
    )(xt)


def _sc_gather(table, idx_flat):
    V, C = table.shape
    R = idx_flat.shape[0]
    info = plsc.get_sparse_core_info()
    nw = info.num_cores * info.num_subcores
    per_w = R // nw
    ch = 128
    n_ch = per_w // ch
    mesh = plsc.VectorSubcoreMesh(core_axis_name="c", subcore_axis_name="s")

    @functools.partial(
        pl.kernel,
        mesh=mesh,
        out_type=jax.ShapeDtypeStruct((R, C), jnp.float32),
        scratch_types=[
            pltpu.VMEM((ch,), jnp.int32),
            pltpu.VMEM((ch,), jnp.int32),
            pltpu.VMEM((ch, C), jnp.float32),
            pltpu.VMEM((ch, C), jnp.float32),
            pltpu.SemaphoreType.DMA,
            pltpu.SemaphoreType.DMA,
        ],
    )
    def gather_kernel(table_hbm, idx_hbm, out_hbm,
                      idx_v0, idx_v1, rows_v0, rows_v1, sem0, sem1):
        wid = lax.axis_index("s") * info.num_cores + lax.axis_index("c")
        base = wid * per_w
        buf = [(idx_v0, rows_v0, sem0), (idx_v1, rows_v1, sem1)]

        def start(j, idx_v, rows_v, sem):
            pltpu.sync_copy(idx_hbm.at[pl.ds(base + j * ch, ch)], idx_v)
            return pltpu.async_copy(table_hbm.at[idx_v], rows_v, sem)

        h = [None, None]
        h[0] = start(0, *buf[0])
        for j in range(n_ch):
            if j + 1 < n_ch:
                h[(j + 1) % 2] = start(j + 1, *buf[(j + 1) % 2])
            h[j % 2].wait()
            idx_v, rows_v, _ = buf[j % 2]
            pltpu.sync_copy(rows_v, out_hbm.at[pl.ds(base + j * ch, ch)])

    return gather_kernel(table, idx_flat)


def _conv_y(feat_ref, xt_ref, wT_ref):
    f32 = jnp.float32
    xi = xt_ref[0]
    c = xi.shape[1]
    xg = feat_ref[0, 0][:, :c]
    e = jnp.concatenate([xg - xi, xi], axis=1)
    return jax.lax.dot_general(e, wT_ref[...], (((1,), (0,)), ((), ())),
                               preferred_element_type=f32)


def _conv_pool_body(feat_ref, xt_ref, wT_ref,
                    ymax_ref, ymin_ref, s1_ref, c1_ref,
                    mx_s, mn_s, s_s, cs_s):
    k = pl.program_id(1)
    y = _conv_y(feat_ref, xt_ref, wT_ref)

    @pl.when(k == 0)
    def _():
        mx_s[...] = y
        mn_s[...] = y
        s_s[...] = y
        cs_s[...] = jnp.zeros(cs_s.shape, jnp.float32)

    @pl.when(k > 0)
    def _():
        mx_s[...] = jnp.maximum(mx_s[...], y)
        mn_s[...] = jnp.minimum(mn_s[...], y)
        t, e = _two_sum(s_s[...], y)
        s_s[...] = t
        cs_s[...] = cs_s[...] + e

    @pl.when(k == _K - 1)
    def _():
        ymax_ref[0] = mx_s[...]
        ymin_ref[0] = mn_s[...]
        hi, co = _comp_reduce(s_s[...], cs_s[...])
        s1_ref[0] = hi
        c1_ref[0] = co


def _var_body(feat_ref, xt_ref, wT_ref, s1_ref, c1_ref,
              v1_ref, cv1_ref, mean_s, v_s, cv_s, *, count):
    k = pl.program_id(1)
    y = _conv_y(feat_ref, xt_ref, wT_ref)

    @pl.when(k == 0)
    def _():
        hi, co = _comp_reduce(
            s1_ref[...].reshape(s1_ref.shape[0], s1_ref.shape[2]),
            c1_ref[...].reshape(c1_ref.shape[0], c1_ref.shape[2]))
        mean_s[...] = (hi + co) / count

    dev = y - mean_s[...]
    sq = dev * dev

    @pl.when(k == 0)
    def _():
        v_s[...] = sq
        cv_s[...] = jnp.zeros(cv_s.shape, jnp.float32)

    @pl.when(k > 0)
    def _():
        t, e = _two_sum(v_s[...], sq)
        v_s[...] = t
        cv_s[...] = cv_s[...] + e

    @pl.when(k == _K - 1)
    def _():
        hi, co = _comp_reduce(v_s[...], cv_s[...])
        v1_ref[0] = hi
        cv1_ref[0] = co


def _bn_body(ymax_ref, ymin_ref, s1_ref, c1_ref, v1_ref, cv1_ref,
             g_ref, b_ref, out_ref, *, count):
    s_hi, s_co = _comp_reduce(
        s1_ref[...].reshape(s1_ref.shape[0], s1_ref.shape[2]),
        c1_ref[...].reshape(c1_ref.shape[0], c1_ref.shape[2]))
    mean = (s_hi + s_co) / count
    v_hi, v_co = _comp_reduce(
        v1_ref[...].reshape(v1_ref.shape[0], v1_ref.shape[2]),
        cv1_ref[...].reshape(cv1_ref.shape[0], cv1_ref.shape[2]))
    var = (v_hi + v_co) / count
    g = g_ref[...]
    ysel = jnp.where(g >= 0.0, ymax_ref[0], ymin_ref[0])
    t = (ysel - mean) / jnp.sqrt(var + 1e-5)
    t = t * g + b_ref[...]
    out_ref[0] = jnp.where(t >= 0.0, t, 0.2 * t)


def _edge_layer(xt, W, g, b):
    B, N, C = xt.shape
    O = W.shape[0]
    f32 = jnp.float32
    wT = jnp.transpose(W)
    count = float(B * N * _K)

    idx = _topk_indices(xt)
    idx_flat = jnp.transpose(idx, (0, 2, 1)).reshape(B * _K * N)
    Cp = max(C, 128)
    table = xt.reshape(B * N, C)
    if Cp != C:
        table = jnp.concatenate(
            [table, jnp.zeros((B * N, Cp - C), f32)], axis=1)
    feat = _sc_gather(table, idx_flat)
    feat = feat.reshape(B, _K, N, Cp)

    feat_spec = pl.BlockSpec((1, 1, N, Cp), lambda i, k: (i, k, 0, 0))
    xt_spec = pl.BlockSpec((1, N, C), lambda i, k: (i, 0, 0))
    w_spec = pl.BlockSpec((2 * C, O), lambda i, k: (0, 0))
    stat_spec = pl.BlockSpec((1, 1, O), lambda i, k: (i, 0, 0))
    stat_all_spec = pl.BlockSpec((B, 1, O), lambda i, k: (0, 0, 0))
    stat_shape = jax.ShapeDtypeStruct((B, 1, O), f32)

    ymax, ymin, s1, c1 = pl.pallas_call(
        _conv_pool_body,
        grid=(B, _K),
        in_specs=[feat_spec, xt_spec, w_spec],
        out_specs=[
            pl.BlockSpec((1, N, O), lambda i, k: (i, 0, 0)),
            pl.BlockSpec((1, N, O), lambda i, k: (i, 0, 0)),
            stat_spec, stat_spec,
        ],
        out_shape=[
            jax.ShapeDtypeStruct((B, N, O), f32),
            jax.ShapeDtypeStruct((B, N, O), f32),
            stat_shape, stat_shape,
        ],
        scratch_shapes=[pltpu.VMEM((N, O), f32)] * 4,
    )(feat, xt, wT)

    v1, cv1 = pl.pallas_call(
        functools.partial(_var_body, count=count),
        grid=(B, _K),
        in_specs=[feat_spec, xt_spec, w_spec, stat_all_spec, stat_all_spec],
        out_specs=[stat_spec, stat_spec],
        out_shape=[stat_shape, stat_shape],
        scratch_shapes=[
            pltpu.VMEM((1, O), f32),
            pltpu.VMEM((N, O), f32),
            pltpu.VMEM((N, O), f32),
        ],
    )(feat, xt, wT, s1, c1)

    return pl.pallas_call(
        functools.partial(_bn_body, count=count),
        grid=(B,),
        in_specs=[
            pl.BlockSpec((1, N, O), lambda i: (i, 0, 0)),
            pl.BlockSpec((1, N, O), lambda i: (i, 0, 0)),
            pl.BlockSpec((B, 1, O), lambda i: (0, 0, 0)),
            pl.BlockSpec((B, 1, O), lambda i: (0, 0, 0)),
            pl.BlockSpec((B, 1, O), lambda i: (0, 0, 0)),
            pl.BlockSpec((B, 1, O), lambda i: (0, 0, 0)),
            pl.BlockSpec((1, O), lambda i: (0, 0)),
            pl.BlockSpec((1, O), lambda i: (0, 0)),
        ],
        out_specs=pl.BlockSpec((1, N, O), lambda i: (i, 0, 0)),
        out_shape=jax.ShapeDtypeStruct((B, N, O), f32),
    )(ymax, ymin, s1, c1, v1, cv1, g.reshape(1, O), b.reshape(1, O))


def kernel(x, W1, W2, W3, W4, g1, b1, g2, b2, g3, b3, g4, b4):
    h = _edge_layer(x, W1, g1, b1)
    h = _edge_layer(h, W2, g2, b2)
    h = _edge_layer(h, W3, g3, b3)
    h = _edge_layer(h, W4, g4, b4)
    return h
